# Initial kernel scaffold; baseline (speedup 1.0000x reference)
import functools

import jax
import jax.numpy as jnp
from jax.experimental import pallas as pl
from jax.experimental.pallas import tpu as pltpu

HIDDEN = 64
HEADS = 2
NG = 64
N = 50000
E = 800000


_PB = 1000


def _pool_mlp_body(h_ref, nt_ref, b_ref, w_refs, acc_num, acc_cnt, out_ref):
    i = pl.program_id(0)

    @pl.when(i == 0)
    def _init():
        acc_num[...] = jnp.zeros_like(acc_num)
        acc_cnt[...] = jnp.zeros_like(acc_cnt)

    h = h_ref[...]
    nt = nt_ref[0]
    b = b_ref[0]
    mask = (nt == 0).astype(jnp.float32)
    gids = jax.lax.broadcasted_iota(jnp.int32, (NG, _PB), 0)
    onehot = (b == gids).astype(jnp.float32)
    hm = h * mask.reshape(_PB, 1)
    acc_num[...] += jnp.dot(onehot, hm, preferred_element_type=jnp.float32)
    acc_cnt[...] += jnp.dot(onehot, mask.reshape(_PB, 1),
                            preferred_element_type=jnp.float32)

    @pl.when(i == pl.num_programs(0) - 1)
    def _final():
        (W1, b1, g1, bb1, W2, b2, g2, bb2, W3, b3, g3, bb3, W4, b4) = [
            r[...] for r in w_refs]

        def ln(v, g, bb):
            m = jnp.mean(v, axis=-1, keepdims=True)
            var = jnp.mean((v - m) ** 2, axis=-1, keepdims=True)
            return (v - m) / jnp.sqrt(var + 1e-5) * g + bb

        def lrelu(v):
            return jnp.maximum(v, 0.2 * v)

        pooled = acc_num[...] / jnp.maximum(acc_cnt[...], 1.0)
        z = lrelu(ln(jnp.dot(pooled, W1, preferred_element_type=jnp.float32) + b1[0], g1[0], bb1[0]))
        z = lrelu(ln(jnp.dot(z, W2, preferred_element_type=jnp.float32) + b2[0], g2[0], bb2[0]))
        z = lrelu(ln(jnp.dot(z, W3, preferred_element_type=jnp.float32) + b3[0], g3[0], bb3[0]))
        z = jnp.dot(z, W4, preferred_element_type=jnp.float32) + b4[0]
        out_ref[...] = jnp.broadcast_to(z.reshape(1, NG), (8, NG))


def _pool_mlp(h, node_type, batch, mp):
    nb = N // _PB
    nt3 = node_type.astype(jnp.int32).reshape(nb, 1, _PB)
    b3 = batch.astype(jnp.int32).reshape(nb, 1, _PB)
    ws = [mp['W1'], mp['b1'].reshape(1, -1), mp['g1'].reshape(1, -1), mp['bb1'].reshape(1, -1),
          mp['W2'], mp['b2'].reshape(1, -1), mp['g2'].reshape(1, -1), mp['bb2'].reshape(1, -1),
          mp['W3'], mp['b3'].reshape(1, -1), mp['g3'].reshape(1, -1), mp['bb3'].reshape(1, -1),
          mp['W4'], mp['b4'].reshape(1, -1)]
    w_specs = [pl.BlockSpec(w.shape, functools.partial(lambda nd, i: (0,) * nd, w.ndim))
               for w in ws]

    grid = (nb,)
    out = pl.pallas_call(
        lambda h_ref, nt_ref, b_ref, *rest: _pool_mlp_body(
            h_ref, nt_ref, b_ref, rest[:-3], rest[-3], rest[-2], rest[-1]),
        grid=grid,
        in_specs=[
            pl.BlockSpec((_PB, HIDDEN), lambda i: (i, 0)),
            pl.BlockSpec((1, 1, _PB), lambda i: (i, 0, 0)),
            pl.BlockSpec((1, 1, _PB), lambda i: (i, 0, 0)),
            *w_specs,
        ],
        out_specs=pl.BlockSpec((8, NG), lambda i: (0, 0)),
        out_shape=jax.ShapeDtypeStruct((8, NG), jnp.float32),
        scratch_shapes=[
            pltpu.VMEM((NG, HIDDEN), jnp.float32),
            pltpu.VMEM((NG, 1), jnp.float32),
        ],
    )(h, nt3, b3, *ws)
    return out[0]


def _ln(v, g, b):
    m = jnp.mean(v, axis=-1, keepdims=True)
    var = jnp.mean((v - m) ** 2, axis=-1, keepdims=True)
    return (v - m) / jnp.sqrt(var + 1e-5) * g + b


def _gatv2(h, edge_index, edge_attr, p):
    src = edge_index[0]
    dst = edge_index[1]
    n = h.shape[0]
    xl = (h @ p['Wl'] + p['bl']).reshape(n, HEADS, HIDDEN)
    xr = (h @ p['Wr'] + p['br']).reshape(n, HEADS, HIDDEN)
    e = (edge_attr @ p['We']).reshape(-1, HEADS, HIDDEN)
    xj = xl[src]
    xi = xr[dst]
    m = jax.nn.leaky_relu(xi + xj + e, 0.2)
    alpha = jnp.sum(m * p['att'][None], axis=-1)
    amax = jax.ops.segment_max(alpha, dst, num_segments=n)
    amax = jnp.where(jnp.isfinite(amax), amax, 0.0)
    ex = jnp.exp(alpha - amax[dst])
    den = jax.ops.segment_sum(ex, dst, num_segments=n)
    a = ex / (den[dst] + 1e-16)
    out = jax.ops.segment_sum(xj * a[..., None], dst, num_segments=n)
    return jnp.mean(out, axis=1) + p['bias']


def kernel(x, node_type, edge_index, edge_attr, batch, params):
    h = x @ params['vp_W'] + params['vp_b'] + params['type_emb'][node_type]
    for lp, nrm in zip(params['layers'], params['norms']):
        hh = jax.nn.leaky_relu(_gatv2(h, edge_index, edge_attr, lp), 0.2)
        h = _ln(h + hh, nrm['g'], nrm['b'])
    return _pool_mlp(h, node_type, batch, params['mlp'])


# speedup vs baseline: 1.0011x; 1.0011x over previous
import functools

import jax
import jax.numpy as jnp
from jax.experimental import pallas as pl
from jax.experimental.pallas import tpu as pltpu

HIDDEN = 64
HEADS = 2
NG = 64
N = 50000
E = 800000


_PB = 1000


def _pool_mlp_body(h_ref, nt_ref, b_ref, w_refs, acc_num, acc_cnt, out_ref):
    i = pl.program_id(0)

    @pl.when(i == 0)
    def _init():
        acc_num[...] = jnp.zeros_like(acc_num)
        acc_cnt[...] = jnp.zeros_like(acc_cnt)

    h = h_ref[...]
    nt = nt_ref[0]
    b = b_ref[0]
    mask = (nt == 0).astype(jnp.float32)
    gids = jax.lax.broadcasted_iota(jnp.int32, (NG, _PB), 0)
    onehot = (b == gids).astype(jnp.float32)
    hm = h * mask.reshape(_PB, 1)
    acc_num[...] += jnp.dot(onehot, hm, preferred_element_type=jnp.float32)
    acc_cnt[...] += jnp.dot(onehot, mask.reshape(_PB, 1),
                            preferred_element_type=jnp.float32)

    @pl.when(i == pl.num_programs(0) - 1)
    def _final():
        (W1, b1, g1, bb1, W2, b2, g2, bb2, W3, b3, g3, bb3, W4, b4) = [
            r[...] for r in w_refs]

        def ln(v, g, bb):
            m = jnp.mean(v, axis=-1, keepdims=True)
            var = jnp.mean((v - m) ** 2, axis=-1, keepdims=True)
            return (v - m) / jnp.sqrt(var + 1e-5) * g + bb

        def lrelu(v):
            return jnp.maximum(v, 0.2 * v)

        pooled = acc_num[...] / jnp.maximum(acc_cnt[...], 1.0)
        z = lrelu(ln(jnp.dot(pooled, W1, preferred_element_type=jnp.float32) + b1[0], g1[0], bb1[0]))
        z = lrelu(ln(jnp.dot(z, W2, preferred_element_type=jnp.float32) + b2[0], g2[0], bb2[0]))
        z = lrelu(ln(jnp.dot(z, W3, preferred_element_type=jnp.float32) + b3[0], g3[0], bb3[0]))
        z = jnp.dot(z, W4, preferred_element_type=jnp.float32) + b4[0]
        out_ref[...] = jnp.broadcast_to(z.reshape(1, NG), (8, NG))


def _pool_mlp(h, node_type, batch, mp):
    nb = N // _PB
    nt3 = node_type.astype(jnp.int32).reshape(nb, 1, _PB)
    b3 = batch.astype(jnp.int32).reshape(nb, 1, _PB)
    ws = [mp['W1'], mp['b1'].reshape(1, -1), mp['g1'].reshape(1, -1), mp['bb1'].reshape(1, -1),
          mp['W2'], mp['b2'].reshape(1, -1), mp['g2'].reshape(1, -1), mp['bb2'].reshape(1, -1),
          mp['W3'], mp['b3'].reshape(1, -1), mp['g3'].reshape(1, -1), mp['bb3'].reshape(1, -1),
          mp['W4'], mp['b4'].reshape(1, -1)]
    w_specs = [pl.BlockSpec(w.shape, functools.partial(lambda nd, i: (0,) * nd, w.ndim))
               for w in ws]

    grid = (nb,)
    out = pl.pallas_call(
        lambda h_ref, nt_ref, b_ref, *rest: _pool_mlp_body(
            h_ref, nt_ref, b_ref, rest[:-3], rest[-2], rest[-1], rest[-3]),
        grid=grid,
        in_specs=[
            pl.BlockSpec((_PB, HIDDEN), lambda i: (i, 0)),
            pl.BlockSpec((1, 1, _PB), lambda i: (i, 0, 0)),
            pl.BlockSpec((1, 1, _PB), lambda i: (i, 0, 0)),
            *w_specs,
        ],
        out_specs=pl.BlockSpec((8, NG), lambda i: (0, 0)),
        out_shape=jax.ShapeDtypeStruct((8, NG), jnp.float32),
        scratch_shapes=[
            pltpu.VMEM((NG, HIDDEN), jnp.float32),
            pltpu.VMEM((NG, 1), jnp.float32),
        ],
    )(h, nt3, b3, *ws)
    return out[0]


def _ln(v, g, b):
    m = jnp.mean(v, axis=-1, keepdims=True)
    var = jnp.mean((v - m) ** 2, axis=-1, keepdims=True)
    return (v - m) / jnp.sqrt(var + 1e-5) * g + b


def _gatv2(h, edge_index, edge_attr, p):
    src = edge_index[0]
    dst = edge_index[1]
    n = h.shape[0]
    xl = (h @ p['Wl'] + p['bl']).reshape(n, HEADS, HIDDEN)
    xr = (h @ p['Wr'] + p['br']).reshape(n, HEADS, HIDDEN)
    e = (edge_attr @ p['We']).reshape(-1, HEADS, HIDDEN)
    xj = xl[src]
    xi = xr[dst]
    m = jax.nn.leaky_relu(xi + xj + e, 0.2)
    alpha = jnp.sum(m * p['att'][None], axis=-1)
    amax = jax.ops.segment_max(alpha, dst, num_segments=n)
    amax = jnp.where(jnp.isfinite(amax), amax, 0.0)
    ex = jnp.exp(alpha - amax[dst])
    den = jax.ops.segment_sum(ex, dst, num_segments=n)
    a = ex / (den[dst] + 1e-16)
    out = jax.ops.segment_sum(xj * a[..., None], dst, num_segments=n)
    return jnp.mean(out, axis=1) + p['bias']


def kernel(x, node_type, edge_index, edge_attr, batch, params):
    h = x @ params['vp_W'] + params['vp_b'] + params['type_emb'][node_type]
    for lp, nrm in zip(params['layers'], params['norms']):
        hh = jax.nn.leaky_relu(_gatv2(h, edge_index, edge_attr, lp), 0.2)
        h = _ln(h + hh, nrm['g'], nrm['b'])
    return _pool_mlp(h, node_type, batch, params['mlp'])
